# baseline (speedup 1.0000x reference)
import functools
import math

import jax
import jax.numpy as jnp
from jax import lax
from jax.experimental import pallas as pl
from jax.experimental.pallas import tpu as pltpu
from jax.experimental.pallas import tpu_sc as plsc

G = 10
NPER = 5000
NP = 5120
NB = NP // 128
K = 10
KP = 16
H = 128
L = 8
ALPHA = 0.1
THETA = 0.5

NTILES = 16
TCHUNK = NP // NTILES
NSUB = NP // 128
GPC = G // 2


SEG = 40
SEGR = NP // SEG
R = 4


def _knn_body(pall_ref, pq_ref, out_ref):
    inf = jnp.float32(jnp.inf)
    pall = pall_ref[0]
    pq = pq_ref[0]
    sqa = jnp.sum(pall * pall, axis=0)
    sqq = jnp.sum(pq * pq, axis=0)
    dots = lax.dot_general(pall, pq, (((0,), (0,)), ((), ())),
                           preferred_element_type=jnp.float32)
    npio = lax.broadcasted_iota(jnp.int32, (1, NP), 1)
    sqa = jnp.where(npio[0] >= NPER, inf, sqa)
    d2b = sqa[:, None] + sqq[None, :] - 2.0 * dots
    jrow = lax.broadcasted_iota(jnp.int32, (NP, 128), 0)
    r = pl.program_id(1)
    qpad = (r * 128 + lax.broadcasted_iota(jnp.int32, (1, 128), 1)) >= NPER

    d2s = d2b.reshape(SEG, SEGR, 128)
    srow = lax.broadcasted_iota(jnp.int32, (SEG, SEGR, 128), 1)
    soff = lax.broadcasted_iota(jnp.int32, (SEG, 1, 128), 0) * SEGR
    cvals, cidxs = [], []
    for t in range(R):
        m = jnp.min(d2s, axis=1, keepdims=True)
        am = jnp.argmin(d2s, axis=1).astype(jnp.int32)[:, None, :]
        if t + 1 < R:
            d2s = jnp.where(srow == am, inf, d2s)
        cvals.append(m)
        cidxs.append(am + soff)
    C = SEG * R
    cval = jnp.concatenate(cvals, axis=1).reshape(C, 128)
    cidx = jnp.concatenate(cidxs, axis=1).reshape(C, 128)

    crow = lax.broadcasted_iota(jnp.int32, (C, 128), 0)
    segio = lax.broadcasted_iota(jnp.int32, (SEG, 128), 0)
    cnt = jnp.zeros((SEG, 128), jnp.int32)
    for t in range(K):
        gm = jnp.argmin(cval, axis=0).astype(jnp.int32)[None, :]
        gi = jnp.min(jnp.where(crow == gm, cidx, NP), axis=0, keepdims=True)
        cval = jnp.where(crow == gm, inf, cval)
        cnt = cnt + jnp.where(segio == gm // R, 1, 0)
        amq = jnp.where(qpad, NP - 1, gi)
        out_ref[0, t] = amq[0]
    dump = jnp.full((128,), NP - 1, jnp.int32)
    for t in range(K, KP):
        out_ref[0, t] = dump

    bad = jnp.any(cnt >= R)

    @pl.when(bad)
    def _fallback():
        d2 = d2b
        for t in range(K):
            m = jnp.min(d2, axis=0, keepdims=True)
            cand = jnp.where(d2 == m, jrow, NP)
            am = jnp.min(cand, axis=0, keepdims=True)
            if t + 1 < K:
                d2 = jnp.where(jrow == am, inf, d2)
            amq = jnp.where(qpad, NP - 1, am)
            out_ref[0, t] = amq[0]


def _knn(posT):
    ng = posT.shape[0]
    return pl.pallas_call(
        _knn_body,
        grid=(ng, NB),
        in_specs=[
            pl.BlockSpec((1, 3, NP), lambda g, r: (g, 0, 0)),
            pl.BlockSpec((1, 3, 128), lambda g, r: (g, 0, r)),
        ],
        out_specs=pl.BlockSpec((1, KP, 128), lambda g, r: (g, 0, r)),
        out_shape=jax.ShapeDtypeStruct((ng, KP, NP), jnp.int32),
    )(posT, posT)


_RB = 512


def _embed_body(pos_ref, w0_ref, b0_ref, out_ref):
    p = pos_ref[...]
    y = lax.dot_general(p, w0_ref[...], (((1,), (1,)), ((), ())),
                        preferred_element_type=jnp.float32)
    out_ref[...] = jnp.maximum(y + b0_ref[...], 0.0)


def _embed(pos_flat, W0, b0r):
    n = pos_flat.shape[0]
    return pl.pallas_call(
        _embed_body,
        grid=(n // _RB,),
        in_specs=[
            pl.BlockSpec((_RB, 3), lambda i: (i, 0)),
            pl.BlockSpec((H, 3), lambda i: (0, 0)),
            pl.BlockSpec((1, H), lambda i: (0, 0)),
        ],
        out_specs=pl.BlockSpec((_RB, H), lambda i: (i, 0)),
        out_shape=jax.ShapeDtypeStruct((n, H), jnp.float32),
    )(pos_flat, W0, b0r)


ZROWS = 64


@functools.cache
def _make_sc_scatter():
    mesh = plsc.VectorSubcoreMesh(core_axis_name="c", subcore_axis_name="s",
                                  num_cores=2, num_subcores=NTILES)
    return functools.partial(
        pl.kernel,
        out_type=jax.ShapeDtypeStruct((2 * NP, H), jnp.float32),
        mesh=mesh,
        scratch_types=[
            pltpu.VMEM((128, H), jnp.float32),
            pltpu.VMEM((KP, 128), jnp.int32),
            pltpu.VMEM((ZROWS, H), jnp.float32),
            pltpu.VMEM_SHARED((NP, H), jnp.float32),
            pltpu.SemaphoreType.DMA,
            pltpu.SemaphoreType.DMA,
        ],
    )(_sc_scatter_body)


def _sc_scatter(x_pair, nbr_pair):
    return _make_sc_scatter()(x_pair, nbr_pair)


def _sc_scatter_body(x_hbm, nbr_hbm, out_hbm, xbuf, idxv, zbuf, aggsh,
                     ssem, zsem):
    c = lax.axis_index("c")
    s = lax.axis_index("s")
    zero16 = jnp.zeros((16,), jnp.float32)

    def _zrow(i, carry):
        for j in range(H // 16):
            zbuf[i, pl.ds(j * 16, 16)] = zero16
        return carry

    lax.fori_loop(0, ZROWS, _zrow, 0)
    zcps = [pltpu.async_copy(zbuf, aggsh.at[pl.ds(s * TCHUNK + z * ZROWS, ZROWS)],
                             zsem) for z in range(TCHUNK // ZROWS)]
    for cp in zcps:
        cp.wait()
    plsc.subcore_barrier()
    nsub_per_tile = (NSUB + NTILES - 1) // NTILES
    for j in range(nsub_per_tile):
        sb = s + NTILES * j

        def _do(sb=sb):
            qbase = c * NP + sb * 128
            pltpu.sync_copy(x_hbm.at[pl.ds(qbase, 128)], xbuf)
            pltpu.sync_copy(nbr_hbm.at[c, :, pl.ds(sb * 128, 128)], idxv)
            cps = [pltpu.async_copy(xbuf, aggsh.at[idxv.at[k]], ssem, add=True)
                   for k in range(K)]
            for cp in cps:
                cp.wait()

        if (j + 1) * NTILES <= NSUB:
            _do()
        else:
            pl.when(sb < NSUB)(_do)
    plsc.subcore_barrier()
    pltpu.sync_copy(aggsh.at[pl.ds(s * TCHUNK, TCHUNK)],
                    out_hbm.at[pl.ds(c * NP + s * TCHUNK, TCHUNK)])


def _layer_body(beta, agg_ref, x0_ref, wc_ref, out_ref):
    h = agg_ref[...] * (1.0 - ALPHA) + ALPHA * x0_ref[...]
    hw = lax.dot_general(h, wc_ref[...], (((1,), (0,)), ((), ())),
                         preferred_element_type=jnp.float32)
    out_ref[...] = jnp.maximum((1.0 - beta) * h + beta * hw, 0.0)


def _layer(agg, x0, Wc_l, beta):
    n = agg.shape[0]
    return pl.pallas_call(
        functools.partial(_layer_body, beta),
        grid=(n // _RB,),
        in_specs=[
            pl.BlockSpec((_RB, H), lambda i: (i, 0)),
            pl.BlockSpec((_RB, H), lambda i: (i, 0)),
            pl.BlockSpec((H, H), lambda i: (0, 0)),
        ],
        out_specs=pl.BlockSpec((_RB, H), lambda i: (i, 0)),
        out_shape=jax.ShapeDtypeStruct((n, H), jnp.float32),
    )(agg, x0, Wc_l)


def _pool_body(x_ref, w1_ref, b1_ref, out_ref):
    x = x_ref[0]
    y = lax.dot_general(x, w1_ref[...], (((1,), (1,)), ((), ())),
                        preferred_element_type=jnp.float32) + b1_ref[...]
    rows = lax.broadcasted_iota(jnp.int32, (NP, 1), 0)
    y = jnp.where(rows < NPER, y, -jnp.float32(jnp.inf))
    out_ref[0] = jnp.max(y, axis=0, keepdims=True)


def _pool(x3, W1, b1r):
    ng = x3.shape[0]
    return pl.pallas_call(
        _pool_body,
        grid=(ng,),
        in_specs=[
            pl.BlockSpec((1, NP, H), lambda g: (g, 0, 0)),
            pl.BlockSpec((H, H), lambda g: (0, 0)),
            pl.BlockSpec((1, H), lambda g: (0, 0)),
        ],
        out_specs=pl.BlockSpec((1, 1, H), lambda g: (g, 0, 0)),
        out_shape=jax.ShapeDtypeStruct((ng, 1, H), jnp.float32),
    )(x3, W1, b1r)


def _head_body(p_ref, wm1_ref, bm1_ref, g1_ref, be1_ref, wm2_ref, bm2_ref,
               g2_ref, be2_ref, wout_ref, bout_ref, out_ref):
    p = p_ref[...]

    def bn(z, gamma, beta_):
        m = jnp.mean(z, axis=0, keepdims=True)
        v = jnp.mean((z - m) * (z - m), axis=0, keepdims=True)
        return (z - m) / jnp.sqrt(v + 1e-5) * gamma + beta_

    c11 = (((1,), (1,)), ((), ()))
    z1 = lax.dot_general(p, wm1_ref[...], c11,
                         preferred_element_type=jnp.float32) + bm1_ref[...]
    h1 = jnp.maximum(bn(z1, g1_ref[...], be1_ref[...]), 0.0)
    z2 = lax.dot_general(h1, wm2_ref[...], c11,
                         preferred_element_type=jnp.float32) + bm2_ref[...]
    h2 = jnp.maximum(bn(z2, g2_ref[...], be2_ref[...]), 0.0)
    logits = lax.dot_general(h2, wout_ref[...], c11,
                             preferred_element_type=jnp.float32) + bout_ref[...]
    mx = jnp.max(logits, axis=1, keepdims=True)
    lse = jnp.log(jnp.sum(jnp.exp(logits - mx), axis=1, keepdims=True)) + mx
    out_ref[...] = logits - lse


def _head(pooled, Wm1, bm1r, g1r, be1r, Wm2, bm2r, g2r, be2r, Wout, boutr):
    full = lambda shape: pl.BlockSpec(shape, lambda: tuple(0 for _ in shape))
    return pl.pallas_call(
        _head_body,
        in_specs=[
            full((G, H)),
            full((H, H)), full((1, H)), full((1, H)), full((1, H)),
            full((H, H)), full((1, H)), full((1, H)), full((1, H)),
            full((10, H)), full((1, 10)),
        ],
        out_specs=full((G, 10)),
        out_shape=jax.ShapeDtypeStruct((G, 10), jnp.float32),
    )(pooled, Wm1, bm1r, g1r, be1r, Wm2, bm2r, g2r, be2r, Wout, boutr)


def kernel(pos, batch, W0, b0, Wc, W1, b1, Wm1, bm1, g1, be1,
           Wm2, bm2, g2, be2, Wout, bout):
    del batch
    f32 = jnp.float32
    pos = pos.astype(f32)
    posg = pos.reshape(G, NPER, 3)
    posp = jnp.pad(posg, ((0, 0), (0, NP - NPER), (0, 0)))
    posT = posp.transpose(0, 2, 1)
    W0 = W0.astype(f32)
    b0r = b0.astype(f32).reshape(1, H)
    W1 = W1.astype(f32)
    b1r = b1.astype(f32).reshape(1, H)
    Wcs = [Wc[l - 1].astype(f32) for l in range(1, L + 1)]
    betas = [float(math.log(THETA / l + 1.0)) for l in range(1, L + 1)]

    NPAIR = G // 2
    nbrs, xs, x0s = [], [], []
    for p in range(NPAIR):
        posT_p = posT[2 * p:2 * p + 2]
        pos_flat_p = posp[2 * p:2 * p + 2].reshape(2 * NP, 3)
        nbrs.append(_knn(posT_p))
        x0s.append(_embed(pos_flat_p, W0, b0r))
    xs = list(x0s)
    for l in range(L):
        aggs = [_sc_scatter(xs[p], nbrs[p]) for p in range(NPAIR)]
        xs = [_layer(aggs[p], x0s[p], Wcs[l], betas[l]) for p in range(NPAIR)]
    pooled_parts = [_pool(xs[p].reshape(2, NP, H), W1, b1r).reshape(2, H)
                    for p in range(NPAIR)]
    pooled = jnp.concatenate(pooled_parts, axis=0)

    out = _head(pooled,
                Wm1.astype(f32), bm1.astype(f32).reshape(1, H),
                g1.astype(f32).reshape(1, H), be1.astype(f32).reshape(1, H),
                Wm2.astype(f32), bm2.astype(f32).reshape(1, H),
                g2.astype(f32).reshape(1, H), be2.astype(f32).reshape(1, H),
                Wout.astype(f32), bout.astype(f32).reshape(1, 10))
    return out


# speedup vs baseline: 1.0001x; 1.0001x over previous
import functools
import math

import jax
import jax.numpy as jnp
from jax import lax
from jax.experimental import pallas as pl
from jax.experimental.pallas import tpu as pltpu
from jax.experimental.pallas import tpu_sc as plsc

G = 10
NPER = 5000
NP = 5120
NB = NP // 128
K = 10
KP = 16
H = 128
L = 8
ALPHA = 0.1
THETA = 0.5

NTILES = 16
TCHUNK = NP // NTILES
NSUB = NP // 128
GPC = G // 2


SEG = 40
SEGR = NP // SEG
R = 4


def _knn_body(pall_ref, pq_ref, out_ref):
    inf = jnp.float32(jnp.inf)
    pall = pall_ref[0]
    pq = pq_ref[0]
    sqa = jnp.sum(pall * pall, axis=0)
    sqq = jnp.sum(pq * pq, axis=0)
    dots = lax.dot_general(pall, pq, (((0,), (0,)), ((), ())),
                           preferred_element_type=jnp.float32)
    npio = lax.broadcasted_iota(jnp.int32, (1, NP), 1)
    sqa = jnp.where(npio[0] >= NPER, inf, sqa)
    d2b = sqa[:, None] + sqq[None, :] - 2.0 * dots
    jrow = lax.broadcasted_iota(jnp.int32, (NP, 128), 0)
    r = pl.program_id(1)
    qpad = (r * 128 + lax.broadcasted_iota(jnp.int32, (1, 128), 1)) >= NPER

    d2s = d2b.reshape(SEG, SEGR, 128)
    srow = lax.broadcasted_iota(jnp.int32, (SEG, SEGR, 128), 1)
    soff = lax.broadcasted_iota(jnp.int32, (SEG, 1, 128), 0) * SEGR
    cvals, cidxs = [], []
    for t in range(R):
        m = jnp.min(d2s, axis=1, keepdims=True)
        am = jnp.argmin(d2s, axis=1).astype(jnp.int32)[:, None, :]
        if t + 1 < R:
            d2s = jnp.where(srow == am, inf, d2s)
        cvals.append(m)
        cidxs.append(am + soff)
    C = SEG * R
    cval = jnp.concatenate(cvals, axis=1).reshape(C, 128)
    cidx = jnp.concatenate(cidxs, axis=1).reshape(C, 128)

    crow = lax.broadcasted_iota(jnp.int32, (C, 128), 0)
    segio = lax.broadcasted_iota(jnp.int32, (SEG, 128), 0)
    cnt = jnp.zeros((SEG, 128), jnp.int32)
    for t in range(K):
        gm = jnp.argmin(cval, axis=0).astype(jnp.int32)[None, :]
        gi = jnp.min(jnp.where(crow == gm, cidx, NP), axis=0, keepdims=True)
        cval = jnp.where(crow == gm, inf, cval)
        cnt = cnt + jnp.where(segio == gm // R, 1, 0)
        amq = jnp.where(qpad, NP - 1, gi)
        out_ref[0, t] = amq[0]
    dump = jnp.full((128,), NP - 1, jnp.int32)
    for t in range(K, KP):
        out_ref[0, t] = dump

    bad = jnp.any(cnt >= R)

    @pl.when(bad)
    def _fallback():
        d2 = d2b
        for t in range(K):
            m = jnp.min(d2, axis=0, keepdims=True)
            cand = jnp.where(d2 == m, jrow, NP)
            am = jnp.min(cand, axis=0, keepdims=True)
            if t + 1 < K:
                d2 = jnp.where(jrow == am, inf, d2)
            amq = jnp.where(qpad, NP - 1, am)
            out_ref[0, t] = amq[0]


def _knn(posT):
    ng = posT.shape[0]
    return pl.pallas_call(
        _knn_body,
        grid=(ng, NB),
        in_specs=[
            pl.BlockSpec((1, 3, NP), lambda g, r: (g, 0, 0)),
            pl.BlockSpec((1, 3, 128), lambda g, r: (g, 0, r)),
        ],
        out_specs=pl.BlockSpec((1, KP, 128), lambda g, r: (g, 0, r)),
        out_shape=jax.ShapeDtypeStruct((ng, KP, NP), jnp.int32),
    )(posT, posT)


_RB = 512


def _embed_body(pos_ref, w0_ref, b0_ref, out_ref):
    p = pos_ref[...]
    y = lax.dot_general(p, w0_ref[...], (((1,), (1,)), ((), ())),
                        preferred_element_type=jnp.float32)
    out_ref[...] = jnp.maximum(y + b0_ref[...], 0.0)


def _embed(pos_flat, W0, b0r):
    n = pos_flat.shape[0]
    return pl.pallas_call(
        _embed_body,
        grid=(n // _RB,),
        in_specs=[
            pl.BlockSpec((_RB, 3), lambda i: (i, 0)),
            pl.BlockSpec((H, 3), lambda i: (0, 0)),
            pl.BlockSpec((1, H), lambda i: (0, 0)),
        ],
        out_specs=pl.BlockSpec((_RB, H), lambda i: (i, 0)),
        out_shape=jax.ShapeDtypeStruct((n, H), jnp.float32),
    )(pos_flat, W0, b0r)


ZROWS = 64


@functools.cache
def _make_sc_scatter():
    mesh = plsc.VectorSubcoreMesh(core_axis_name="c", subcore_axis_name="s",
                                  num_cores=2, num_subcores=NTILES)
    return functools.partial(
        pl.kernel,
        out_type=jax.ShapeDtypeStruct((2 * NP, H), jnp.float32),
        mesh=mesh,
        scratch_types=[
            pltpu.VMEM((128, H), jnp.float32),
            pltpu.VMEM((KP, 128), jnp.int32),
            pltpu.VMEM((ZROWS, H), jnp.float32),
            pltpu.VMEM_SHARED((NP, H), jnp.float32),
            pltpu.SemaphoreType.DMA,
            pltpu.SemaphoreType.DMA,
        ],
    )(_sc_scatter_body)


def _sc_scatter(x_pair, nbr_pair):
    return _make_sc_scatter()(x_pair, nbr_pair)


def _sc_scatter_body(x_hbm, nbr_hbm, out_hbm, xbuf, idxv, zbuf, aggsh,
                     ssem, zsem):
    c = lax.axis_index("c")
    s = lax.axis_index("s")
    zero16 = jnp.zeros((16,), jnp.float32)

    def _zrow(i, carry):
        for j in range(H // 16):
            zbuf[i, pl.ds(j * 16, 16)] = zero16
        return carry

    lax.fori_loop(0, ZROWS, _zrow, 0)
    zcps = [pltpu.async_copy(zbuf, aggsh.at[pl.ds(s * TCHUNK + z * ZROWS, ZROWS)],
                             zsem) for z in range(TCHUNK // ZROWS)]
    for cp in zcps:
        cp.wait()
    plsc.subcore_barrier()
    nsub_per_tile = (NSUB + NTILES - 1) // NTILES
    for j in range(nsub_per_tile):
        sb = s + NTILES * j

        def _do(sb=sb):
            qbase = c * NP + sb * 128
            pltpu.sync_copy(x_hbm.at[pl.ds(qbase, 128)], xbuf)
            pltpu.sync_copy(nbr_hbm.at[c, :, pl.ds(sb * 128, 128)], idxv)
            cps = [pltpu.async_copy(xbuf, aggsh.at[idxv.at[k]], ssem, add=True)
                   for k in range(K)]
            for cp in cps:
                cp.wait()

        if (j + 1) * NTILES <= NSUB:
            _do()
        else:
            pl.when(sb < NSUB)(_do)
    plsc.subcore_barrier()
    pltpu.sync_copy(aggsh.at[pl.ds(s * TCHUNK, TCHUNK)],
                    out_hbm.at[pl.ds(c * NP + s * TCHUNK, TCHUNK)])


def _layer_body(beta, agg_ref, x0_ref, wc_ref, out_ref):
    h = agg_ref[...] * (1.0 - ALPHA) + ALPHA * x0_ref[...]
    hw = lax.dot_general(h, wc_ref[...], (((1,), (0,)), ((), ())),
                         preferred_element_type=jnp.float32)
    out_ref[...] = jnp.maximum((1.0 - beta) * h + beta * hw, 0.0)


def _layer(agg, x0, Wc_l, beta):
    n = agg.shape[0]
    return pl.pallas_call(
        functools.partial(_layer_body, beta),
        grid=(n // _RB,),
        in_specs=[
            pl.BlockSpec((_RB, H), lambda i: (i, 0)),
            pl.BlockSpec((_RB, H), lambda i: (i, 0)),
            pl.BlockSpec((H, H), lambda i: (0, 0)),
        ],
        out_specs=pl.BlockSpec((_RB, H), lambda i: (i, 0)),
        out_shape=jax.ShapeDtypeStruct((n, H), jnp.float32),
    )(agg, x0, Wc_l)


def _pool_body(x_ref, w1_ref, b1_ref, out_ref):
    x = x_ref[0]
    y = lax.dot_general(x, w1_ref[...], (((1,), (1,)), ((), ())),
                        preferred_element_type=jnp.float32) + b1_ref[...]
    rows = lax.broadcasted_iota(jnp.int32, (NP, 1), 0)
    y = jnp.where(rows < NPER, y, -jnp.float32(jnp.inf))
    out_ref[0] = jnp.max(y, axis=0, keepdims=True)


def _pool(x3, W1, b1r):
    ng = x3.shape[0]
    return pl.pallas_call(
        _pool_body,
        grid=(ng,),
        in_specs=[
            pl.BlockSpec((1, NP, H), lambda g: (g, 0, 0)),
            pl.BlockSpec((H, H), lambda g: (0, 0)),
            pl.BlockSpec((1, H), lambda g: (0, 0)),
        ],
        out_specs=pl.BlockSpec((1, 1, H), lambda g: (g, 0, 0)),
        out_shape=jax.ShapeDtypeStruct((ng, 1, H), jnp.float32),
    )(x3, W1, b1r)


def _head_body(p_ref, wm1_ref, bm1_ref, g1_ref, be1_ref, wm2_ref, bm2_ref,
               g2_ref, be2_ref, wout_ref, bout_ref, out_ref):
    p = p_ref[...]

    def bn(z, gamma, beta_):
        m = jnp.mean(z, axis=0, keepdims=True)
        v = jnp.mean((z - m) * (z - m), axis=0, keepdims=True)
        return (z - m) / jnp.sqrt(v + 1e-5) * gamma + beta_

    c11 = (((1,), (1,)), ((), ()))
    z1 = lax.dot_general(p, wm1_ref[...], c11,
                         preferred_element_type=jnp.float32) + bm1_ref[...]
    h1 = jnp.maximum(bn(z1, g1_ref[...], be1_ref[...]), 0.0)
    z2 = lax.dot_general(h1, wm2_ref[...], c11,
                         preferred_element_type=jnp.float32) + bm2_ref[...]
    h2 = jnp.maximum(bn(z2, g2_ref[...], be2_ref[...]), 0.0)
    logits = lax.dot_general(h2, wout_ref[...], c11,
                             preferred_element_type=jnp.float32) + bout_ref[...]
    mx = jnp.max(logits, axis=1, keepdims=True)
    lse = jnp.log(jnp.sum(jnp.exp(logits - mx), axis=1, keepdims=True)) + mx
    out_ref[...] = logits - lse


def _head(pooled, Wm1, bm1r, g1r, be1r, Wm2, bm2r, g2r, be2r, Wout, boutr):
    full = lambda shape: pl.BlockSpec(shape, lambda: tuple(0 for _ in shape))
    return pl.pallas_call(
        _head_body,
        in_specs=[
            full((G, H)),
            full((H, H)), full((1, H)), full((1, H)), full((1, H)),
            full((H, H)), full((1, H)), full((1, H)), full((1, H)),
            full((10, H)), full((1, 10)),
        ],
        out_specs=full((G, 10)),
        out_shape=jax.ShapeDtypeStruct((G, 10), jnp.float32),
    )(pooled, Wm1, bm1r, g1r, be1r, Wm2, bm2r, g2r, be2r, Wout, boutr)


def kernel(pos, batch, W0, b0, Wc, W1, b1, Wm1, bm1, g1, be1,
           Wm2, bm2, g2, be2, Wout, bout):
    del batch
    f32 = jnp.float32
    pos = pos.astype(f32)
    posg = pos.reshape(G, NPER, 3)
    posp = jnp.pad(posg, ((0, 0), (0, NP - NPER), (0, 0)))
    posT = posp.transpose(0, 2, 1)
    W0 = W0.astype(f32)
    b0r = b0.astype(f32).reshape(1, H)
    W1 = W1.astype(f32)
    b1r = b1.astype(f32).reshape(1, H)
    Wcs = [Wc[l - 1].astype(f32) for l in range(1, L + 1)]
    betas = [float(math.log(THETA / l + 1.0)) for l in range(1, L + 1)]

    NPAIR = G // 2
    nbrs, xs, x0s = [None] * NPAIR, [None] * NPAIR, [None] * NPAIR
    done = [0] * NPAIR

    def _advance(p, upto):
        while done[p] < upto:
            l = done[p]
            agg = _sc_scatter(xs[p], nbrs[p])
            xs[p] = _layer(agg, x0s[p], Wcs[l], betas[l])
            done[p] += 1

    for p in range(NPAIR):
        posT_p = posT[2 * p:2 * p + 2]
        pos_flat_p = posp[2 * p:2 * p + 2].reshape(2 * NP, 3)
        nbrs[p] = _knn(posT_p)
        xs[p] = x0s[p] = _embed(pos_flat_p, W0, b0r)
        for q in range(p):
            _advance(q, min(L, 3 * (p - q)))
    for l in range(1, L + 1):
        for p in range(NPAIR):
            _advance(p, l)
    pooled_parts = [_pool(xs[p].reshape(2, NP, H), W1, b1r).reshape(2, H)
                    for p in range(NPAIR)]
    pooled = jnp.concatenate(pooled_parts, axis=0)

    out = _head(pooled,
                Wm1.astype(f32), bm1.astype(f32).reshape(1, H),
                g1.astype(f32).reshape(1, H), be1.astype(f32).reshape(1, H),
                Wm2.astype(f32), bm2.astype(f32).reshape(1, H),
                g2.astype(f32).reshape(1, H), be2.astype(f32).reshape(1, H),
                Wout.astype(f32), bout.astype(f32).reshape(1, 10))
    return out
